# baseline (speedup 1.0000x reference)
import functools

import jax
import jax.numpy as jnp
from jax import lax
from jax.experimental import pallas as pl
from jax.experimental.pallas import tpu as pltpu
from jax.experimental.pallas import tpu_sc as plsc

N = 16 * 512
C = 81
L = 16
NW = 16
PW = N // NW
EW = PW * 4
G = 128
NG = EW // G


def _body(tb_hbm, cls_hbm, pb_hbm, out_hbm,
          cls_v, bidx_v, idx_v, pmask_v, tgt_v, rows_v, accbuf, shpart,
          red_v, out_v, sem):
    sid = lax.axis_index("s")
    base = sid * PW

    pltpu.sync_copy(cls_hbm.at[pl.ds(base, PW)], cls_v)
    pltpu.sync_copy(tb_hbm.at[pl.ds(base * 4, EW)], tgt_v)

    iota = lax.iota(jnp.int32, L)
    zero = jnp.zeros((L,), jnp.float32)

    for t in range(PW // L):
        c = cls_v[pl.ds(L * t, L)]
        safe = jnp.clip(c, 0, C - 1)
        gi = ((base + L * t + iota) * C + safe) * 4
        bidx_v[pl.ds(L * t, L)] = gi
        pmask_v[pl.ds(L * t, L)] = jnp.where(c > 0, 1.0, 0.0).astype(jnp.float32)

    rowoff = iota // 4
    coloff = iota & 3
    for j in range(EW // L):
        b = plsc.load_gather(bidx_v, [j * 4 + rowoff])
        idx_v[j // (G // L), pl.ds((j % (G // L)) * L, L)] = b + coloff

    copies = [
        pltpu.async_copy(pb_hbm.at[idx_v.at[k]], rows_v.at[pl.ds(k * G, G)], sem)
        for k in range(NG)
    ]
    for cp in copies:
        cp.wait()

    def chunk(j, carry):
        acc, cnt = carry
        m = plsc.load_gather(pmask_v, [j * 4 + rowoff])
        p = rows_v[pl.ds(j * L, L)]
        t16 = tgt_v[pl.ds(j * L, L)]
        d = jnp.abs(t16 - p)
        e = jnp.where(d < 1.0, (0.5 * d) * d, d - 0.5)
        return acc + e * m, cnt + m

    acc, cnt = lax.fori_loop(0, EW // L, chunk, (zero, zero))

    accbuf[0, :] = acc
    accbuf[1, :] = cnt
    pltpu.sync_copy(accbuf, shpart.at[sid])
    plsc.subcore_barrier()

    @pl.when(sid == 0)
    def _():
        pltpu.sync_copy(shpart, red_v)
        tot = zero
        cn4 = zero
        for i in range(NW):
            tot = tot + red_v[i, 0, :]
            cn4 = cn4 + red_v[i, 1, :]
        totv = jnp.full((L,), jnp.sum(tot), jnp.float32)
        c4v = jnp.full((L,), jnp.sum(cn4), jnp.float32)
        out_v[...] = jnp.where(c4v > 0.0, totv / jnp.maximum(c4v, 1.0),
                               jnp.zeros((L,), jnp.float32))
        pltpu.sync_copy(out_v, out_hbm)


@jax.jit
def _bbox_loss_sc(tb, cls, pb):
    mesh = plsc.VectorSubcoreMesh(
        core_axis_name="c", subcore_axis_name="s",
        num_cores=1, num_subcores=NW)
    k = pl.kernel(
        _body,
        out_type=jax.ShapeDtypeStruct((L,), jnp.float32),
        mesh=mesh,
        compiler_params=pltpu.CompilerParams(
            use_tc_tiling_on_sc=False, needs_layout_passes=False),
        scratch_types=[
            pltpu.VMEM((PW,), jnp.int32),
            pltpu.VMEM((PW,), jnp.int32),
            pltpu.VMEM((NG, G), jnp.int32),
            pltpu.VMEM((PW,), jnp.float32),
            pltpu.VMEM((EW,), jnp.float32),
            pltpu.VMEM((EW,), jnp.float32),
            pltpu.VMEM((2, L), jnp.float32),
            pltpu.VMEM_SHARED((NW, 2, L), jnp.float32),
            pltpu.VMEM((NW, 2, L), jnp.float32),
            pltpu.VMEM((L,), jnp.float32),
            pltpu.SemaphoreType.DMA,
        ],
    )
    return k(tb, cls, pb)


def kernel(target_bbox, target_class_ids, pred_bbox):
    tb = target_bbox.reshape(-1)
    cls = target_class_ids.reshape(-1)
    pb = pred_bbox.reshape(-1)
    out = _bbox_loss_sc(tb, cls, pb)
    return out[0]


# speedup vs baseline: 37.9640x; 37.9640x over previous
import jax
import jax.numpy as jnp
from jax import lax
from jax.experimental import pallas as pl
from jax.experimental.pallas import tpu as pltpu
from jax.experimental.pallas import tpu_sc as plsc

B = 16
R = 512
N = B * R
C = 81
L = 16
NW = 16
PW = N // NW
EW = PW * 4
G = 128
NG = EW // G
RT = R // 128


def _body(tb_hbm, cls_hbm, pb_hbm, out_hbm,
          cls_v, idx_v, pmask_v, tgt_v, rows_v, accbuf, shpart,
          red_v, out_v, sem):
    sid = lax.axis_index("s")

    cbase = (sid // 8) * (RT * 1024) + (sid % 8) * 128
    for rt in range(RT):
        pltpu.sync_copy(cls_hbm.at[pl.ds(cbase + rt * 1024, 128)],
                        cls_v.at[pl.ds(rt * 128, 128)])
    pltpu.sync_copy(tb_hbm.at[pl.ds(sid * EW, EW)], tgt_v)

    iota = lax.iota(jnp.int32, L)
    zero = jnp.zeros((L,), jnp.float32)

    kb = sid * C * 16
    for rt in range(RT):
        for q in range(128 // L):
            k = cls_v[pl.ds(rt * 128 + q * L, L)]
            safe = jnp.clip(k, 0, C - 1)
            pmask_v[pl.ds(rt * 128 + q * L, L)] = (
                jnp.where(k > 0, 1.0, 0.0).astype(jnp.float32))
            ebase = (kb + safe * 16 + rt * 4) * 128 + q * L + iota
            for c in range(4):
                idx_v[rt * 4 + c, pl.ds(q * L, L)] = ebase + c * 128

    copies = [
        pltpu.async_copy(pb_hbm.at[idx_v.at[j]], rows_v.at[pl.ds(j * G, G)], sem)
        for j in range(NG)
    ]
    for cp in copies:
        cp.wait()

    def chunk(j, carry):
        acc, cnt = carry
        rcq = j * L
        m = pmask_v[pl.ds((j // 32) * 128 + (j & 7) * L, L)]
        p = rows_v[pl.ds(rcq, L)]
        t16 = tgt_v[pl.ds(rcq, L)]
        d = jnp.abs(t16 - p)
        e = jnp.where(d < 1.0, (0.5 * d) * d, d - 0.5)
        return acc + e * m, cnt + m

    acc, cnt = lax.fori_loop(0, EW // L, chunk, (zero, zero))

    accbuf[0, :] = acc
    accbuf[1, :] = cnt
    pltpu.sync_copy(accbuf, shpart.at[sid])
    plsc.subcore_barrier()

    @pl.when(sid == 0)
    def _():
        pltpu.sync_copy(shpart, red_v)
        tot = zero
        cn4 = zero
        for i in range(NW):
            tot = tot + red_v[i, 0, :]
            cn4 = cn4 + red_v[i, 1, :]
        totv = jnp.full((L,), jnp.sum(tot), jnp.float32)
        c4v = jnp.full((L,), jnp.sum(cn4), jnp.float32)
        out_v[...] = jnp.where(c4v > 0.0, totv / jnp.maximum(c4v, 1.0),
                               jnp.zeros((L,), jnp.float32))
        pltpu.sync_copy(out_v, out_hbm)


@jax.jit
def _bbox_loss_sc(tb, cls, pb):
    mesh = plsc.VectorSubcoreMesh(
        core_axis_name="c", subcore_axis_name="s",
        num_cores=1, num_subcores=NW)
    k = pl.kernel(
        _body,
        out_type=jax.ShapeDtypeStruct((L,), jnp.float32),
        mesh=mesh,
        compiler_params=pltpu.CompilerParams(
            use_tc_tiling_on_sc=False, needs_layout_passes=False),
        scratch_types=[
            pltpu.VMEM((PW,), jnp.int32),
            pltpu.VMEM((NG, G), jnp.int32),
            pltpu.VMEM((PW,), jnp.float32),
            pltpu.VMEM((EW,), jnp.float32),
            pltpu.VMEM((EW,), jnp.float32),
            pltpu.VMEM((2, L), jnp.float32),
            pltpu.VMEM_SHARED((NW, 2, L), jnp.float32),
            pltpu.VMEM((NW, 2, L), jnp.float32),
            pltpu.VMEM((L,), jnp.float32),
            pltpu.SemaphoreType.DMA,
        ],
    )
    return k(tb, cls, pb)


def kernel(target_bbox, target_class_ids, pred_bbox):
    tb = (target_bbox.reshape(B, RT, 128, 4)
          .transpose(0, 1, 3, 2).reshape(-1))
    cls = (target_class_ids.reshape(B // 8, 8, RT, 128)
           .transpose(0, 2, 1, 3).reshape(-1))
    pb = (pred_bbox.reshape(B, RT, 128, C, 4)
          .transpose(0, 3, 1, 4, 2).reshape(-1))
    out = _bbox_loss_sc(tb, cls, pb)
    return out[0]


# speedup vs baseline: 39.0993x; 1.0299x over previous
import jax
import jax.numpy as jnp
from jax import lax
from jax.experimental import pallas as pl
from jax.experimental.pallas import tpu as pltpu
from jax.experimental.pallas import tpu_sc as plsc

B = 16
R = 512
N = B * R
C = 81
L = 16
NW = 16
PW = N // NW
EW = PW * 4
G = 128
NG = EW // G
RT = R // 128


def _body(tb_hbm, cls_hbm, pb_hbm, out_hbm,
          cls_v, idx_v, pmask_v, tgt_v, rows_v, accbuf, shpart,
          red_v, out_v, tsem, gsem0, gsem1, gsem2, gsem3):
    sid = lax.axis_index("s")
    gsems = [gsem0, gsem1, gsem2, gsem3]

    cbase = (sid // 8) * (RT * 1024) + (sid % 8) * 128
    for rt in range(RT):
        pltpu.sync_copy(cls_hbm.at[pl.ds(cbase + rt * 1024, 128)],
                        cls_v.at[pl.ds(rt * 128, 128)])
    tcopy = pltpu.async_copy(tb_hbm.at[pl.ds(sid * EW, EW)], tgt_v, tsem)

    iota = lax.iota(jnp.int32, L)
    zero = jnp.zeros((L,), jnp.float32)

    kb = sid * C * 16
    copies = []
    for rt in range(RT):
        for q in range(128 // L):
            k = cls_v[pl.ds(rt * 128 + q * L, L)]
            safe = jnp.clip(k, 0, C - 1)
            pmask_v[pl.ds(rt * 128 + q * L, L)] = (
                jnp.where(k > 0, 1.0, 0.0).astype(jnp.float32))
            ebase = (kb + safe * 16 + rt * 4) * 128 + q * L + iota
            for c in range(4):
                idx_v[rt * 4 + c, pl.ds(q * L, L)] = ebase + c * 128
        copies.append([
            pltpu.async_copy(pb_hbm.at[idx_v.at[rt * 4 + c]],
                             rows_v.at[pl.ds((rt * 4 + c) * G, G)],
                             gsems[rt])
            for c in range(4)
        ])
    tcopy.wait()

    acc = zero
    cnt = zero
    for rt in range(RT):
        for cp in copies[rt]:
            cp.wait()
        for c in range(4):
            for q in range(128 // L):
                off = (rt * 4 + c) * G + q * L
                m = pmask_v[pl.ds(rt * 128 + q * L, L)]
                p = rows_v[pl.ds(off, L)]
                t16 = tgt_v[pl.ds(off, L)]
                d = jnp.abs(t16 - p)
                e = jnp.where(d < 1.0, (0.5 * d) * d, d - 0.5)
                acc = acc + e * m
                cnt = cnt + m

    accbuf[0, :] = acc
    accbuf[1, :] = cnt
    pltpu.sync_copy(accbuf, shpart.at[sid])
    plsc.subcore_barrier()

    @pl.when(sid == 0)
    def _():
        pltpu.sync_copy(shpart, red_v)
        tot = zero
        cn4 = zero
        for i in range(NW):
            tot = tot + red_v[i, 0, :]
            cn4 = cn4 + red_v[i, 1, :]
        totv = jnp.full((L,), jnp.sum(tot), jnp.float32)
        c4v = jnp.full((L,), jnp.sum(cn4), jnp.float32)
        out_v[...] = jnp.where(c4v > 0.0, totv / jnp.maximum(c4v, 1.0),
                               jnp.zeros((L,), jnp.float32))
        pltpu.sync_copy(out_v, out_hbm)


@jax.jit
def _bbox_loss_sc(tb, cls, pb):
    mesh = plsc.VectorSubcoreMesh(
        core_axis_name="c", subcore_axis_name="s",
        num_cores=1, num_subcores=NW)
    k = pl.kernel(
        _body,
        out_type=jax.ShapeDtypeStruct((L,), jnp.float32),
        mesh=mesh,
        compiler_params=pltpu.CompilerParams(
            use_tc_tiling_on_sc=False, needs_layout_passes=False),
        scratch_types=[
            pltpu.VMEM((PW,), jnp.int32),
            pltpu.VMEM((NG, G), jnp.int32),
            pltpu.VMEM((PW,), jnp.float32),
            pltpu.VMEM((EW,), jnp.float32),
            pltpu.VMEM((EW,), jnp.float32),
            pltpu.VMEM((2, L), jnp.float32),
            pltpu.VMEM_SHARED((NW, 2, L), jnp.float32),
            pltpu.VMEM((NW, 2, L), jnp.float32),
            pltpu.VMEM((L,), jnp.float32),
            pltpu.SemaphoreType.DMA,
            pltpu.SemaphoreType.DMA,
            pltpu.SemaphoreType.DMA,
            pltpu.SemaphoreType.DMA,
            pltpu.SemaphoreType.DMA,
        ],
    )
    return k(tb, cls, pb)


def kernel(target_bbox, target_class_ids, pred_bbox):
    tb = (target_bbox.reshape(B, RT, 128, 4)
          .transpose(0, 1, 3, 2).reshape(-1))
    cls = (target_class_ids.reshape(B // 8, 8, RT, 128)
           .transpose(0, 2, 1, 3).reshape(-1))
    pb = (pred_bbox.reshape(B, RT, 128, C, 4)
          .transpose(0, 3, 1, 4, 2).reshape(-1))
    out = _bbox_loss_sc(tb, cls, pb)
    return out[0]


# speedup vs baseline: 41.2488x; 1.0550x over previous
import jax
import jax.numpy as jnp
from jax import lax
from jax.experimental import pallas as pl
from jax.experimental.pallas import tpu as pltpu
from jax.experimental.pallas import tpu_sc as plsc

B = 16
R = 512
N = B * R
C = 81
L = 16
NW = 16
PW = N // NW
EW = PW * 4
G = 128
NG = EW // G
RT = R // 128


def _body(tb_hbm, cls_hbm, pb_hbm, out_hbm,
          cls_v, idx_v, pmask_v, tgt_v, rows_v, accbuf, shpart,
          red_v, out_v, tsem, gsem0, gsem1, gsem2, gsem3):
    sid = lax.axis_index("s")
    gsems = [gsem0, gsem1, gsem2, gsem3]

    cbase = (sid // 8) * (RT * 1024) + (sid % 8) * 128
    ccopies = [
        pltpu.async_copy(cls_hbm.at[pl.ds(cbase + rt * 1024, 128)],
                         cls_v.at[pl.ds(rt * 128, 128)], gsems[rt])
        for rt in range(RT)
    ]
    tcopy = pltpu.async_copy(tb_hbm.at[pl.ds(sid * EW, EW)], tgt_v, tsem)

    iota = lax.iota(jnp.int32, L)
    zero = jnp.zeros((L,), jnp.float32)

    kb = sid * C * 16
    copies = []
    for rt in range(RT):
        ccopies[rt].wait()
        for q in range(128 // L):
            k = cls_v[pl.ds(rt * 128 + q * L, L)]
            safe = jnp.clip(k, 0, C - 1)
            pmask_v[pl.ds(rt * 128 + q * L, L)] = (
                jnp.where(k > 0, 1.0, 0.0).astype(jnp.float32))
            ebase = (kb + safe * 16 + rt * 4) * 128 + q * L + iota
            for c in range(4):
                idx_v[rt * 4 + c, pl.ds(q * L, L)] = ebase + c * 128
        copies.append([
            pltpu.async_copy(pb_hbm.at[idx_v.at[rt * 4 + c]],
                             rows_v.at[pl.ds((rt * 4 + c) * G, G)],
                             gsems[rt])
            for c in range(4)
        ])
    tcopy.wait()

    acc = zero
    cnt = zero
    for rt in range(RT):
        for cp in copies[rt]:
            cp.wait()
        for c in range(4):
            for q in range(128 // L):
                off = (rt * 4 + c) * G + q * L
                m = pmask_v[pl.ds(rt * 128 + q * L, L)]
                p = rows_v[pl.ds(off, L)]
                t16 = tgt_v[pl.ds(off, L)]
                d = jnp.abs(t16 - p)
                e = jnp.where(d < 1.0, (0.5 * d) * d, d - 0.5)
                acc = acc + e * m
                cnt = cnt + m

    accbuf[0, :] = acc
    accbuf[1, :] = cnt
    pltpu.sync_copy(accbuf, shpart.at[sid])
    plsc.subcore_barrier()

    @pl.when(sid == 0)
    def _():
        pltpu.sync_copy(shpart, red_v)
        tot = zero
        cn4 = zero
        for i in range(NW):
            tot = tot + red_v[i, 0, :]
            cn4 = cn4 + red_v[i, 1, :]
        totv = jnp.full((L,), jnp.sum(tot), jnp.float32)
        c4v = jnp.full((L,), jnp.sum(cn4), jnp.float32)
        out_v[...] = jnp.where(c4v > 0.0, totv / jnp.maximum(c4v, 1.0),
                               jnp.zeros((L,), jnp.float32))
        pltpu.sync_copy(out_v, out_hbm)


@jax.jit
def _bbox_loss_sc(tb, cls, pb):
    mesh = plsc.VectorSubcoreMesh(
        core_axis_name="c", subcore_axis_name="s",
        num_cores=1, num_subcores=NW)
    k = pl.kernel(
        _body,
        out_type=jax.ShapeDtypeStruct((L,), jnp.float32),
        mesh=mesh,
        compiler_params=pltpu.CompilerParams(
            use_tc_tiling_on_sc=False, needs_layout_passes=False),
        scratch_types=[
            pltpu.VMEM((PW,), jnp.int32),
            pltpu.VMEM((NG, G), jnp.int32),
            pltpu.VMEM((PW,), jnp.float32),
            pltpu.VMEM((EW,), jnp.float32),
            pltpu.VMEM((EW,), jnp.float32),
            pltpu.VMEM((2, L), jnp.float32),
            pltpu.VMEM_SHARED((NW, 2, L), jnp.float32),
            pltpu.VMEM((NW, 2, L), jnp.float32),
            pltpu.VMEM((L,), jnp.float32),
            pltpu.SemaphoreType.DMA,
            pltpu.SemaphoreType.DMA,
            pltpu.SemaphoreType.DMA,
            pltpu.SemaphoreType.DMA,
            pltpu.SemaphoreType.DMA,
        ],
    )
    return k(tb, cls, pb)


def kernel(target_bbox, target_class_ids, pred_bbox):
    tb = (target_bbox.reshape(B, RT, 128, 4)
          .transpose(0, 1, 3, 2).reshape(-1))
    cls = (target_class_ids.reshape(B // 8, 8, RT, 128)
           .transpose(0, 2, 1, 3).reshape(-1))
    pb = (pred_bbox.reshape(B, RT, 128, C, 4)
          .transpose(0, 3, 1, 4, 2).reshape(-1))
    out = _bbox_loss_sc(tb, cls, pb)
    return out[0]


# speedup vs baseline: 41.6507x; 1.0097x over previous
import jax
import jax.numpy as jnp
from jax import lax
from jax.experimental import pallas as pl
from jax.experimental.pallas import tpu as pltpu
from jax.experimental.pallas import tpu_sc as plsc

B = 16
R = 512
N = B * R
C = 81
L = 16
NW = 16
PW = N // NW
EW = PW * 4
G = 128
NG = EW // G
RT = R // 128


def _body(tb_hbm, cls_hbm, pb_hbm, out_hbm,
          cls_v, idx_v, pmask_v, tgt_v, rows_v, accbuf, shpart,
          red_v, out_v, tsem, gsem0, gsem1, gsem2, gsem3):
    sid = lax.axis_index("s")
    gsems = [gsem0, gsem1, gsem2, gsem3]

    cbase = (sid // 8) * (RT * 1024) + (sid % 8) * 128
    ccopies = [
        pltpu.async_copy(cls_hbm.at[pl.ds(cbase + rt * 1024, 128)],
                         cls_v.at[pl.ds(rt * 128, 128)], gsems[rt])
        for rt in range(RT)
    ]
    tcopy = pltpu.async_copy(tb_hbm.at[pl.ds(sid * EW, EW)], tgt_v, tsem)

    iota = lax.iota(jnp.int32, L)
    zero = jnp.zeros((L,), jnp.float32)

    kb = sid * C * 16
    copies = []
    cntp = zero
    for rt in range(RT):
        ccopies[rt].wait()
        for q in range(128 // L):
            k = cls_v[pl.ds(rt * 128 + q * L, L)]
            safe = jnp.clip(k, 0, C - 1)
            pm = jnp.where(k > 0, 1.0, 0.0).astype(jnp.float32)
            pmask_v[pl.ds(rt * 128 + q * L, L)] = pm
            cntp = cntp + pm
            ebase = (kb + safe * 16 + rt * 4) * 128 + q * L + iota
            for c in range(4):
                idx_v[rt * 4 + c, pl.ds(q * L, L)] = ebase + c * 128
        copies.append([
            pltpu.async_copy(pb_hbm.at[idx_v.at[rt * 4 + c]],
                             rows_v.at[pl.ds((rt * 4 + c) * G, G)],
                             gsems[rt])
            for c in range(4)
        ])
    tcopy.wait()

    acc = zero
    cnt = cntp * 4.0
    for rt in range(RT):
        for cp in copies[rt]:
            cp.wait()
        for q in range(128 // L):
            m = pmask_v[pl.ds(rt * 128 + q * L, L)]
            for c in range(4):
                off = (rt * 4 + c) * G + q * L
                p = rows_v[pl.ds(off, L)]
                t16 = tgt_v[pl.ds(off, L)]
                d = jnp.abs(t16 - p)
                e = jnp.where(d < 1.0, (0.5 * d) * d, d - 0.5)
                acc = acc + e * m

    accbuf[0, :] = acc
    accbuf[1, :] = cnt
    pltpu.sync_copy(accbuf, shpart.at[sid])
    plsc.subcore_barrier()

    @pl.when(sid == 0)
    def _():
        pltpu.sync_copy(shpart, red_v)
        tot = zero
        cn4 = zero
        for i in range(NW):
            tot = tot + red_v[i, 0, :]
            cn4 = cn4 + red_v[i, 1, :]
        totv = jnp.full((L,), jnp.sum(tot), jnp.float32)
        c4v = jnp.full((L,), jnp.sum(cn4), jnp.float32)
        out_v[...] = jnp.where(c4v > 0.0, totv / jnp.maximum(c4v, 1.0),
                               jnp.zeros((L,), jnp.float32))
        pltpu.sync_copy(out_v, out_hbm)


@jax.jit
def _bbox_loss_sc(tb, cls, pb):
    mesh = plsc.VectorSubcoreMesh(
        core_axis_name="c", subcore_axis_name="s",
        num_cores=1, num_subcores=NW)
    k = pl.kernel(
        _body,
        out_type=jax.ShapeDtypeStruct((L,), jnp.float32),
        mesh=mesh,
        compiler_params=pltpu.CompilerParams(
            use_tc_tiling_on_sc=False, needs_layout_passes=False),
        scratch_types=[
            pltpu.VMEM((PW,), jnp.int32),
            pltpu.VMEM((NG, G), jnp.int32),
            pltpu.VMEM((PW,), jnp.float32),
            pltpu.VMEM((EW,), jnp.float32),
            pltpu.VMEM((EW,), jnp.float32),
            pltpu.VMEM((2, L), jnp.float32),
            pltpu.VMEM_SHARED((NW, 2, L), jnp.float32),
            pltpu.VMEM((NW, 2, L), jnp.float32),
            pltpu.VMEM((L,), jnp.float32),
            pltpu.SemaphoreType.DMA,
            pltpu.SemaphoreType.DMA,
            pltpu.SemaphoreType.DMA,
            pltpu.SemaphoreType.DMA,
            pltpu.SemaphoreType.DMA,
        ],
    )
    return k(tb, cls, pb)


def kernel(target_bbox, target_class_ids, pred_bbox):
    tb = (target_bbox.reshape(B, RT, 128, 4)
          .transpose(0, 1, 3, 2).reshape(-1))
    cls = (target_class_ids.reshape(B // 8, 8, RT, 128)
           .transpose(0, 2, 1, 3).reshape(-1))
    pb = (pred_bbox.reshape(B, RT, 128, C, 4)
          .transpose(0, 3, 1, 4, 2).reshape(-1))
    out = _bbox_loss_sc(tb, cls, pb)
    return out[0]
